# baseline (speedup 1.0000x reference)
import functools

import jax
import jax.numpy as jnp
from jax import lax
from jax.experimental import pallas as pl
from jax.experimental.pallas import tpu as pltpu
from jax.experimental.pallas import tpu_sc as plsc

N = 10000
E = 320000
D = 128
NC = 2
NS = 16
NW = NC * NS
CHUNK = 64
CW = 160
IH = 32
NBUF = 4
IH_DEG = 80
E_PAD = NW * CW * CHUNK
NPAD = E_PAD - E
NDUM = 16
NACC = 10240
ZSTRIPE = NACC // NS
R = 1024

def _sc_degree_body(dst_hbm, ones_hbm, zeros_hbm, out_hbm, idx_v, ones_v, acc):
  c = lax.axis_index("c")
  s = lax.axis_index("s")
  w = c * NS + s
  pltpu.sync_copy(zeros_hbm, acc.at[pl.ds(s * ZSTRIPE, ZSTRIPE)])
  pltpu.sync_copy(ones_hbm, ones_v)
  for p in range(CW // IH_DEG):
    pltpu.sync_copy(dst_hbm.at[pl.ds(w * CW + p * IH_DEG, IH_DEG), :], idx_v)
    if p == 0:
      plsc.subcore_barrier()

    def body(i, carry):
      pltpu.sync_copy(ones_v, acc.at[idx_v.at[i]], add=True)
      return carry

    lax.fori_loop(0, IH_DEG, body, 0)
  plsc.subcore_barrier()
  pltpu.sync_copy(acc.at[pl.ds(s * ZSTRIPE, ZSTRIPE)],
                  out_hbm.at[c, pl.ds(s * ZSTRIPE, ZSTRIPE)])


@functools.cache
def _get_sc_degree():
  mesh = plsc.VectorSubcoreMesh(
      core_axis_name="c", subcore_axis_name="s", num_cores=NC,
      num_subcores=NS)
  return pl.kernel(
      _sc_degree_body,
      out_type=jax.ShapeDtypeStruct((NC, NACC), jnp.float32),
      mesh=mesh,
      scratch_types=[
          pltpu.VMEM((IH_DEG, CHUNK), jnp.int32),
          pltpu.VMEM((CHUNK,), jnp.float32),
          pltpu.VMEM_SHARED((NACC,), jnp.float32),
      ],
  )


def _sc_scatter_body(src_hbm, dst_hbm, m_hbm, zeros_hbm, out_hbm,
                     src_v, dst_v, rows_v, acc, sem0, sem1, sem2, sem3):
  sems = [sem0, sem1, sem2, sem3]
  c = lax.axis_index("c")
  s = lax.axis_index("s")
  w = c * NS + s
  pltpu.sync_copy(zeros_hbm, rows_v.at[0])
  for z in range(ZSTRIPE // CHUNK):
    pltpu.sync_copy(rows_v.at[0],
                    acc.at[pl.ds(s * ZSTRIPE + z * CHUNK, CHUNK), :])

  for p in range(CW // IH):
    base = w * CW + p * IH
    pltpu.sync_copy(src_hbm.at[pl.ds(base, IH), :], src_v)
    pltpu.sync_copy(dst_hbm.at[pl.ds(base, IH), :], dst_v)
    if p == 0:
      plsc.subcore_barrier()

    for b in range(NBUF):
      pltpu.async_copy(m_hbm.at[src_v.at[b]], rows_v.at[b], sems[b])

    def body(i, carry):
      for b in range(NBUF):
        ci = NBUF * i + b
        pltpu.make_async_copy(m_hbm.at[src_v.at[ci]], rows_v.at[b],
                              sems[b]).wait()
        pltpu.sync_copy(rows_v.at[b], acc.at[dst_v.at[ci]], add=True)

        @pl.when(ci + NBUF < IH)
        def _():
          pltpu.async_copy(m_hbm.at[src_v.at[ci + NBUF]], rows_v.at[b],
                           sems[b])
      return carry

    lax.fori_loop(0, IH // NBUF, body, 0)
  plsc.subcore_barrier()
  pltpu.sync_copy(acc.at[pl.ds(s * ZSTRIPE, ZSTRIPE), :],
                  out_hbm.at[c, pl.ds(s * ZSTRIPE, ZSTRIPE), :])


@functools.cache
def _get_sc_scatter():
  mesh = plsc.VectorSubcoreMesh(
      core_axis_name="c", subcore_axis_name="s", num_cores=NC,
      num_subcores=NS)
  return pl.kernel(
      _sc_scatter_body,
      out_type=jax.ShapeDtypeStruct((NC, NACC, D), jnp.float32),
      mesh=mesh,
      scratch_types=[
          pltpu.VMEM((IH, CHUNK), jnp.int32),
          pltpu.VMEM((IH, CHUNK), jnp.int32),
          pltpu.VMEM((NBUF, CHUNK, D), jnp.float32),
          pltpu.VMEM_SHARED((NACC, D), jnp.float32),
          pltpu.SemaphoreType.DMA,
          pltpu.SemaphoreType.DMA,
          pltpu.SemaphoreType.DMA,
          pltpu.SemaphoreType.DMA,
      ],
  )


def _tc_linear_body(x_ref, w0_ref, b0_ref, wn0_ref, bn0_ref, deg_ref,
                    h0_ref, nb_ref, m_ref):
  i = pl.program_id(0)
  xb = x_ref[...]
  h0 = lax.dot_general(xb, w0_ref[...], (((1,), (1,)), ((), ())),
                       preferred_element_type=jnp.float32) + b0_ref[...]
  nb = lax.dot_general(h0, wn0_ref[...], (((1,), (1,)), ((), ())),
                       preferred_element_type=jnp.float32) + bn0_ref[...]
  deg = deg_ref[0, pl.ds(i * R, R)] + deg_ref[1, pl.ds(i * R, R)] + 1.0
  h0_ref[...] = h0
  nb_ref[...] = nb
  m_ref[...] = nb * lax.rsqrt(deg)[:, None]


def _tc_linear(x, w0, b0, wn0, bn0, deg2):
  return pl.pallas_call(
      _tc_linear_body,
      grid=(pl.cdiv(N, R),),
      in_specs=[
          pl.BlockSpec((R, D), lambda i: (i, 0)),
          pl.BlockSpec((D, D), lambda i: (0, 0)),
          pl.BlockSpec((1, D), lambda i: (0, 0)),
          pl.BlockSpec((D, D), lambda i: (0, 0)),
          pl.BlockSpec((1, D), lambda i: (0, 0)),
          pl.BlockSpec((NC, NACC), lambda i: (0, 0)),
      ],
      out_specs=[
          pl.BlockSpec((R, D), lambda i: (i, 0)),
          pl.BlockSpec((R, D), lambda i: (i, 0)),
          pl.BlockSpec((R, D), lambda i: (i, 0)),
      ],
      out_shape=[
          jax.ShapeDtypeStruct((N, D), jnp.float32),
          jax.ShapeDtypeStruct((N, D), jnp.float32),
          jax.ShapeDtypeStruct((N, D), jnp.float32),
      ],
  )(x, w0, b0, wn0, bn0, deg2)


def _tc_final_body(h0_ref, nb_ref, m_ref, s_ref, deg_ref, w1_ref, b1_ref,
                   out_ref):
  i = pl.program_id(0)
  deg = deg_ref[0, pl.ds(i * R, R)] + deg_ref[1, pl.ds(i * R, R)] + 1.0
  dinv = lax.rsqrt(deg)[:, None]
  agg = dinv * (s_ref[0] + s_ref[1] + m_ref[...])
  h = h0_ref[...] + 0.5 * agg + 0.5 * nb_ref[...]
  nrm = jnp.sqrt(jnp.sum(h * h, axis=1, keepdims=True))
  h = h / jnp.maximum(nrm, 1e-12)
  h = jnp.maximum(h, 0.0)
  out_ref[...] = lax.dot_general(h, w1_ref[...], (((1,), (1,)), ((), ())),
                                 preferred_element_type=jnp.float32
                                 ) + b1_ref[...]


def _tc_final(h0, nb, m, s2, deg2, w1, b1):
  return pl.pallas_call(
      _tc_final_body,
      grid=(pl.cdiv(N, R),),
      in_specs=[
          pl.BlockSpec((R, D), lambda i: (i, 0)),
          pl.BlockSpec((R, D), lambda i: (i, 0)),
          pl.BlockSpec((R, D), lambda i: (i, 0)),
          pl.BlockSpec((NC, R, D), lambda i: (0, i, 0)),
          pl.BlockSpec((NC, NACC), lambda i: (0, 0)),
          pl.BlockSpec((D, D), lambda i: (0, 0)),
          pl.BlockSpec((1, D), lambda i: (0, 0)),
      ],
      out_specs=pl.BlockSpec((R, D), lambda i: (i, 0)),
      out_shape=jax.ShapeDtypeStruct((N, D), jnp.float32),
  )(h0, nb, m, s2, deg2, w1, b1)


@jax.jit
def kernel(x, edge_index, W0, b0, Wn0, bn0, W1, b1):
  src = edge_index[0].astype(jnp.int32)
  dst = edge_index[1].astype(jnp.int32)
  ar = jnp.arange(NPAD, dtype=jnp.int32)
  src_p = jnp.concatenate([src, ar % N]).reshape(NW * CW, CHUNK)
  dst_p = jnp.concatenate([dst, N + (ar % NDUM)]).reshape(NW * CW, CHUNK)
  ones1 = jnp.ones((CHUNK,), jnp.float32)
  zeros1 = jnp.zeros((ZSTRIPE,), jnp.float32)
  zeros2 = jnp.zeros((CHUNK, D), jnp.float32)

  deg2 = _get_sc_degree()(dst_p, ones1, zeros1)
  h0, nb, m = _tc_linear(x, W0, b0.reshape(1, D), Wn0, bn0.reshape(1, D),
                         deg2)
  s2 = _get_sc_scatter()(src_p, dst_p, m, zeros2)
  return _tc_final(h0, nb, m, s2, deg2, W1, b1.reshape(1, D))


# speedup vs baseline: 1.0131x; 1.0131x over previous
import functools

import jax
import jax.numpy as jnp
from jax import lax
from jax.experimental import pallas as pl
from jax.experimental.pallas import tpu as pltpu
from jax.experimental.pallas import tpu_sc as plsc

N = 10000
E = 320000
D = 128
NC = 2
NS = 16
NW = NC * NS
CHUNK = 64
CW = 160
IH = 32
NBUF = 4
IH_DEG = 80
E_PAD = NW * CW * CHUNK
NPAD = E_PAD - E
NDUM = 16
NACC = 10240
ZSTRIPE = NACC // NS
R = 1024

def _sc_degree_body(dst_hbm, ones_hbm, zeros_hbm, out_hbm, idx_v, ones_v, acc):
  c = lax.axis_index("c")
  s = lax.axis_index("s")
  w = c * NS + s
  pltpu.sync_copy(zeros_hbm, acc.at[pl.ds(s * ZSTRIPE, ZSTRIPE)])
  pltpu.sync_copy(ones_hbm, ones_v)
  for p in range(CW // IH_DEG):
    pltpu.sync_copy(dst_hbm.at[pl.ds(w * CW + p * IH_DEG, IH_DEG), :], idx_v)
    if p == 0:
      plsc.subcore_barrier()

    def body(i, carry):
      pltpu.sync_copy(ones_v, acc.at[idx_v.at[i]], add=True)
      return carry

    lax.fori_loop(0, IH_DEG, body, 0)
  plsc.subcore_barrier()
  pltpu.sync_copy(acc.at[pl.ds(s * ZSTRIPE, ZSTRIPE)],
                  out_hbm.at[c, pl.ds(s * ZSTRIPE, ZSTRIPE)])


@functools.cache
def _get_sc_degree():
  mesh = plsc.VectorSubcoreMesh(
      core_axis_name="c", subcore_axis_name="s", num_cores=NC,
      num_subcores=NS)
  return pl.kernel(
      _sc_degree_body,
      out_type=jax.ShapeDtypeStruct((NC, NACC), jnp.float32),
      mesh=mesh,
      scratch_types=[
          pltpu.VMEM((IH_DEG, CHUNK), jnp.int32),
          pltpu.VMEM((CHUNK,), jnp.float32),
          pltpu.VMEM_SHARED((NACC,), jnp.float32),
      ],
  )


def _sc_scatter_body(src_hbm, dst_hbm, m_hbm, zeros_hbm, out_hbm,
                     src_v, dst_v, rows_v, acc, sem0, sem1, sem2, sem3):
  sems = [sem0, sem1, sem2, sem3]
  c = lax.axis_index("c")
  s = lax.axis_index("s")
  w = c * NS + s
  pltpu.sync_copy(zeros_hbm, rows_v.at[0])
  for z in range(ZSTRIPE // CHUNK):
    pltpu.sync_copy(rows_v.at[0],
                    acc.at[pl.ds(s * ZSTRIPE + z * CHUNK, CHUNK), :])

  for p in range(CW // IH):
    base = w * CW + p * IH
    pltpu.sync_copy(src_hbm.at[pl.ds(base, IH), :], src_v)
    pltpu.sync_copy(dst_hbm.at[pl.ds(base, IH), :], dst_v)
    if p == 0:
      plsc.subcore_barrier()

    for b in range(NBUF):
      pltpu.async_copy(m_hbm.at[src_v.at[b]], rows_v.at[b], sems[b])

    def body(i, carry):
      for b in range(NBUF):
        ci = NBUF * i + b
        pltpu.make_async_copy(m_hbm.at[src_v.at[ci]], rows_v.at[b],
                              sems[b]).wait()
        pltpu.sync_copy(rows_v.at[b], acc.at[dst_v.at[ci]], add=True)

        @pl.when(ci + NBUF < IH)
        def _():
          pltpu.async_copy(m_hbm.at[src_v.at[ci + NBUF]], rows_v.at[b],
                           sems[b])
      return carry

    lax.fori_loop(0, IH // NBUF, body, 0)
  plsc.subcore_barrier()
  pltpu.sync_copy(acc.at[pl.ds(s * ZSTRIPE, ZSTRIPE), :],
                  out_hbm.at[c, pl.ds(s * ZSTRIPE, ZSTRIPE), :])


@functools.cache
def _get_sc_scatter():
  mesh = plsc.VectorSubcoreMesh(
      core_axis_name="c", subcore_axis_name="s", num_cores=NC,
      num_subcores=NS)
  return pl.kernel(
      _sc_scatter_body,
      out_type=jax.ShapeDtypeStruct((NC, NACC, D), jnp.float32),
      mesh=mesh,
      scratch_types=[
          pltpu.VMEM((IH, CHUNK), jnp.int32),
          pltpu.VMEM((IH, CHUNK), jnp.int32),
          pltpu.VMEM((NBUF, CHUNK, D), jnp.float32),
          pltpu.VMEM_SHARED((NACC, D), jnp.float32),
          pltpu.SemaphoreType.DMA,
          pltpu.SemaphoreType.DMA,
          pltpu.SemaphoreType.DMA,
          pltpu.SemaphoreType.DMA,
      ],
  )


def _tc_linear_body(x_ref, w0_ref, b0_ref, wn0_ref, bn0_ref,
                    h0_ref, nb_ref):
  xb = x_ref[...]
  h0 = lax.dot_general(xb, w0_ref[...], (((1,), (1,)), ((), ())),
                       preferred_element_type=jnp.float32) + b0_ref[...]
  nb = lax.dot_general(h0, wn0_ref[...], (((1,), (1,)), ((), ())),
                       preferred_element_type=jnp.float32) + bn0_ref[...]
  h0_ref[...] = h0
  nb_ref[...] = nb


def _tc_linear(x, w0, b0, wn0, bn0):
  return pl.pallas_call(
      _tc_linear_body,
      grid=(pl.cdiv(N, R),),
      in_specs=[
          pl.BlockSpec((R, D), lambda i: (i, 0)),
          pl.BlockSpec((D, D), lambda i: (0, 0)),
          pl.BlockSpec((1, D), lambda i: (0, 0)),
          pl.BlockSpec((D, D), lambda i: (0, 0)),
          pl.BlockSpec((1, D), lambda i: (0, 0)),
      ],
      out_specs=[
          pl.BlockSpec((R, D), lambda i: (i, 0)),
          pl.BlockSpec((R, D), lambda i: (i, 0)),
      ],
      out_shape=[
          jax.ShapeDtypeStruct((N, D), jnp.float32),
          jax.ShapeDtypeStruct((N, D), jnp.float32),
      ],
  )(x, w0, b0, wn0, bn0)


def _tc_scale_body(nb_ref, deg_ref, m_ref):
  i = pl.program_id(0)
  deg = deg_ref[0, pl.ds(i * R, R)] + deg_ref[1, pl.ds(i * R, R)] + 1.0
  m_ref[...] = nb_ref[...] * lax.rsqrt(deg)[:, None]


def _tc_scale(nb, deg2):
  return pl.pallas_call(
      _tc_scale_body,
      grid=(pl.cdiv(N, R),),
      in_specs=[
          pl.BlockSpec((R, D), lambda i: (i, 0)),
          pl.BlockSpec((NC, NACC), lambda i: (0, 0)),
      ],
      out_specs=pl.BlockSpec((R, D), lambda i: (i, 0)),
      out_shape=jax.ShapeDtypeStruct((N, D), jnp.float32),
  )(nb, deg2)


def _tc_final_body(h0_ref, nb_ref, m_ref, s_ref, deg_ref, w1_ref, b1_ref,
                   out_ref):
  i = pl.program_id(0)
  deg = deg_ref[0, pl.ds(i * R, R)] + deg_ref[1, pl.ds(i * R, R)] + 1.0
  dinv = lax.rsqrt(deg)[:, None]
  agg = dinv * (s_ref[0] + s_ref[1] + m_ref[...])
  h = h0_ref[...] + 0.5 * agg + 0.5 * nb_ref[...]
  nrm = jnp.sqrt(jnp.sum(h * h, axis=1, keepdims=True))
  h = h / jnp.maximum(nrm, 1e-12)
  h = jnp.maximum(h, 0.0)
  out_ref[...] = lax.dot_general(h, w1_ref[...], (((1,), (1,)), ((), ())),
                                 preferred_element_type=jnp.float32
                                 ) + b1_ref[...]


def _tc_final(h0, nb, m, s2, deg2, w1, b1):
  return pl.pallas_call(
      _tc_final_body,
      grid=(pl.cdiv(N, R),),
      in_specs=[
          pl.BlockSpec((R, D), lambda i: (i, 0)),
          pl.BlockSpec((R, D), lambda i: (i, 0)),
          pl.BlockSpec((R, D), lambda i: (i, 0)),
          pl.BlockSpec((NC, R, D), lambda i: (0, i, 0)),
          pl.BlockSpec((NC, NACC), lambda i: (0, 0)),
          pl.BlockSpec((D, D), lambda i: (0, 0)),
          pl.BlockSpec((1, D), lambda i: (0, 0)),
      ],
      out_specs=pl.BlockSpec((R, D), lambda i: (i, 0)),
      out_shape=jax.ShapeDtypeStruct((N, D), jnp.float32),
  )(h0, nb, m, s2, deg2, w1, b1)


@jax.jit
def kernel(x, edge_index, W0, b0, Wn0, bn0, W1, b1):
  src = edge_index[0].astype(jnp.int32)
  dst = edge_index[1].astype(jnp.int32)
  ar = jnp.arange(NPAD, dtype=jnp.int32)
  src_p = jnp.concatenate([src, ar % N]).reshape(NW * CW, CHUNK)
  dst_p = jnp.concatenate([dst, N + (ar % NDUM)]).reshape(NW * CW, CHUNK)
  ones1 = jnp.ones((CHUNK,), jnp.float32)
  zeros1 = jnp.zeros((ZSTRIPE,), jnp.float32)
  zeros2 = jnp.zeros((CHUNK, D), jnp.float32)

  deg2 = _get_sc_degree()(dst_p, ones1, zeros1)
  h0, nb = _tc_linear(x, W0, b0.reshape(1, D), Wn0, bn0.reshape(1, D))
  m = _tc_scale(nb, deg2)
  s2 = _get_sc_scatter()(src_p, dst_p, m, zeros2)
  return _tc_final(h0, nb, m, s2, deg2, W1, b1.reshape(1, D))


# speedup vs baseline: 1.0540x; 1.0404x over previous
import functools

import jax
import jax.numpy as jnp
from jax import lax
from jax.experimental import pallas as pl
from jax.experimental.pallas import tpu as pltpu
from jax.experimental.pallas import tpu_sc as plsc

N = 10000
E = 320000
D = 128
NC = 2
NS = 16
NW = NC * NS
CHUNK = 64
CW = 160
IH = 32
NBUF = 4
CHUNK_DEG = 128
CW_DEG = 80
IH_DEG = 16
E_PAD = NW * CW * CHUNK
NPAD = E_PAD - E
NDUM = 16
NACC = 10240
ZSTRIPE = NACC // NS
R = 1024

def _sc_degree_body(dst_hbm, ones_hbm, zeros_hbm, out_hbm, idx_v, ones_v, acc):
  c = lax.axis_index("c")
  s = lax.axis_index("s")
  w = c * NS + s
  pltpu.sync_copy(zeros_hbm, acc.at[pl.ds(s * ZSTRIPE, ZSTRIPE)])
  pltpu.sync_copy(ones_hbm, ones_v)
  for p in range(CW_DEG // IH_DEG):
    pltpu.sync_copy(dst_hbm.at[pl.ds(w * CW_DEG + p * IH_DEG, IH_DEG), :],
                    idx_v)
    if p == 0:
      plsc.subcore_barrier()

    def body(i, carry):
      pltpu.sync_copy(ones_v, acc.at[idx_v.at[i]], add=True)
      return carry

    lax.fori_loop(0, IH_DEG, body, 0)
  plsc.subcore_barrier()
  pltpu.sync_copy(acc.at[pl.ds(s * ZSTRIPE, ZSTRIPE)],
                  out_hbm.at[c, pl.ds(s * ZSTRIPE, ZSTRIPE)])


@functools.cache
def _get_sc_degree():
  mesh = plsc.VectorSubcoreMesh(
      core_axis_name="c", subcore_axis_name="s", num_cores=NC,
      num_subcores=NS)
  return pl.kernel(
      _sc_degree_body,
      out_type=jax.ShapeDtypeStruct((NC, NACC), jnp.float32),
      mesh=mesh,
      scratch_types=[
          pltpu.VMEM((IH_DEG, CHUNK_DEG), jnp.int32),
          pltpu.VMEM((CHUNK_DEG,), jnp.float32),
          pltpu.VMEM_SHARED((NACC,), jnp.float32),
      ],
  )


def _sc_scatter_body(src_hbm, dst_hbm, m_hbm, zeros_hbm, out_hbm,
                     src_v, dst_v, rows_v, acc, sem0, sem1, sem2, sem3,
                     sem_ix):
  sems = [sem0, sem1, sem2, sem3]
  c = lax.axis_index("c")
  s = lax.axis_index("s")
  w = c * NS + s
  pltpu.sync_copy(zeros_hbm, rows_v.at[0])
  for z in range(ZSTRIPE // CHUNK):
    pltpu.sync_copy(rows_v.at[0],
                    acc.at[pl.ds(s * ZSTRIPE + z * CHUNK, CHUNK), :])

  base = w * CW
  nph = CW // IH
  pltpu.sync_copy(src_hbm.at[pl.ds(base, IH), :], src_v.at[0])
  pltpu.sync_copy(dst_hbm.at[pl.ds(base, IH), :], dst_v.at[0])
  plsc.subcore_barrier()
  for p in range(nph):
    pb = p % 2
    if p + 1 < nph:
      pltpu.async_copy(src_hbm.at[pl.ds(base + (p + 1) * IH, IH), :],
                       src_v.at[1 - pb], sem_ix)
      pltpu.async_copy(dst_hbm.at[pl.ds(base + (p + 1) * IH, IH), :],
                       dst_v.at[1 - pb], sem_ix)

    for b in range(NBUF):
      pltpu.async_copy(m_hbm.at[src_v.at[pb, b]], rows_v.at[b], sems[b])

    def body(i, carry):
      for b in range(NBUF):
        ci = NBUF * i + b
        pltpu.make_async_copy(m_hbm.at[src_v.at[pb, ci]], rows_v.at[b],
                              sems[b]).wait()
        pltpu.sync_copy(rows_v.at[b], acc.at[dst_v.at[pb, ci]], add=True)

        @pl.when(ci + NBUF < IH)
        def _():
          pltpu.async_copy(m_hbm.at[src_v.at[pb, ci + NBUF]], rows_v.at[b],
                           sems[b])
      return carry

    lax.fori_loop(0, IH // NBUF, body, 0)
    if p + 1 < nph:
      pltpu.make_async_copy(src_hbm.at[pl.ds(base, IH), :],
                            src_v.at[1 - pb], sem_ix).wait()
      pltpu.make_async_copy(dst_hbm.at[pl.ds(base, IH), :],
                            dst_v.at[1 - pb], sem_ix).wait()
  plsc.subcore_barrier()
  pltpu.sync_copy(acc.at[pl.ds(s * ZSTRIPE, ZSTRIPE), :],
                  out_hbm.at[c, pl.ds(s * ZSTRIPE, ZSTRIPE), :])


@functools.cache
def _get_sc_scatter():
  mesh = plsc.VectorSubcoreMesh(
      core_axis_name="c", subcore_axis_name="s", num_cores=NC,
      num_subcores=NS)
  return pl.kernel(
      _sc_scatter_body,
      out_type=jax.ShapeDtypeStruct((NC, NACC, D), jnp.float32),
      mesh=mesh,
      scratch_types=[
          pltpu.VMEM((2, IH, CHUNK), jnp.int32),
          pltpu.VMEM((2, IH, CHUNK), jnp.int32),
          pltpu.VMEM((NBUF, CHUNK, D), jnp.float32),
          pltpu.VMEM_SHARED((NACC, D), jnp.float32),
          pltpu.SemaphoreType.DMA,
          pltpu.SemaphoreType.DMA,
          pltpu.SemaphoreType.DMA,
          pltpu.SemaphoreType.DMA,
          pltpu.SemaphoreType.DMA,
      ],
  )


def _tc_linear_body(x_ref, w0_ref, b0_ref, wn0_ref, bn0_ref,
                    h0_ref, nb_ref):
  xb = x_ref[...]
  h0 = lax.dot_general(xb, w0_ref[...], (((1,), (1,)), ((), ())),
                       preferred_element_type=jnp.float32) + b0_ref[...]
  nb = lax.dot_general(h0, wn0_ref[...], (((1,), (1,)), ((), ())),
                       preferred_element_type=jnp.float32) + bn0_ref[...]
  h0_ref[...] = h0
  nb_ref[...] = nb


def _tc_linear(x, w0, b0, wn0, bn0):
  return pl.pallas_call(
      _tc_linear_body,
      grid=(pl.cdiv(N, R),),
      in_specs=[
          pl.BlockSpec((R, D), lambda i: (i, 0)),
          pl.BlockSpec((D, D), lambda i: (0, 0)),
          pl.BlockSpec((1, D), lambda i: (0, 0)),
          pl.BlockSpec((D, D), lambda i: (0, 0)),
          pl.BlockSpec((1, D), lambda i: (0, 0)),
      ],
      out_specs=[
          pl.BlockSpec((R, D), lambda i: (i, 0)),
          pl.BlockSpec((R, D), lambda i: (i, 0)),
      ],
      out_shape=[
          jax.ShapeDtypeStruct((N, D), jnp.float32),
          jax.ShapeDtypeStruct((N, D), jnp.float32),
      ],
  )(x, w0, b0, wn0, bn0)


def _tc_scale_body(nb_ref, deg_ref, m_ref):
  i = pl.program_id(0)
  deg = deg_ref[0, pl.ds(i * R, R)] + deg_ref[1, pl.ds(i * R, R)] + 1.0
  m_ref[...] = nb_ref[...] * lax.rsqrt(deg)[:, None]


def _tc_scale(nb, deg2):
  return pl.pallas_call(
      _tc_scale_body,
      grid=(pl.cdiv(N, R),),
      in_specs=[
          pl.BlockSpec((R, D), lambda i: (i, 0)),
          pl.BlockSpec((NC, NACC), lambda i: (0, 0)),
      ],
      out_specs=pl.BlockSpec((R, D), lambda i: (i, 0)),
      out_shape=jax.ShapeDtypeStruct((N, D), jnp.float32),
  )(nb, deg2)


def _tc_final_body(h0_ref, nb_ref, m_ref, s_ref, deg_ref, w1_ref, b1_ref,
                   out_ref):
  i = pl.program_id(0)
  deg = deg_ref[0, pl.ds(i * R, R)] + deg_ref[1, pl.ds(i * R, R)] + 1.0
  dinv = lax.rsqrt(deg)[:, None]
  agg = dinv * (s_ref[0] + s_ref[1] + m_ref[...])
  h = h0_ref[...] + 0.5 * agg + 0.5 * nb_ref[...]
  nrm = jnp.sqrt(jnp.sum(h * h, axis=1, keepdims=True))
  h = h / jnp.maximum(nrm, 1e-12)
  h = jnp.maximum(h, 0.0)
  out_ref[...] = lax.dot_general(h, w1_ref[...], (((1,), (1,)), ((), ())),
                                 preferred_element_type=jnp.float32
                                 ) + b1_ref[...]


def _tc_final(h0, nb, m, s2, deg2, w1, b1):
  return pl.pallas_call(
      _tc_final_body,
      grid=(pl.cdiv(N, R),),
      in_specs=[
          pl.BlockSpec((R, D), lambda i: (i, 0)),
          pl.BlockSpec((R, D), lambda i: (i, 0)),
          pl.BlockSpec((R, D), lambda i: (i, 0)),
          pl.BlockSpec((NC, R, D), lambda i: (0, i, 0)),
          pl.BlockSpec((NC, NACC), lambda i: (0, 0)),
          pl.BlockSpec((D, D), lambda i: (0, 0)),
          pl.BlockSpec((1, D), lambda i: (0, 0)),
      ],
      out_specs=pl.BlockSpec((R, D), lambda i: (i, 0)),
      out_shape=jax.ShapeDtypeStruct((N, D), jnp.float32),
  )(h0, nb, m, s2, deg2, w1, b1)


@jax.jit
def kernel(x, edge_index, W0, b0, Wn0, bn0, W1, b1):
  src = edge_index[0].astype(jnp.int32)
  dst = edge_index[1].astype(jnp.int32)
  ar = jnp.arange(NPAD, dtype=jnp.int32)
  src_p = jnp.concatenate([src, ar % N]).reshape(NW * CW, CHUNK)
  dst_f = jnp.concatenate([dst, N + (ar % NDUM)])
  dst_p = dst_f.reshape(NW * CW, CHUNK)
  dst_pd = dst_f.reshape(NW * CW_DEG, CHUNK_DEG)
  ones1 = jnp.ones((CHUNK_DEG,), jnp.float32)
  zeros1 = jnp.zeros((ZSTRIPE,), jnp.float32)
  zeros2 = jnp.zeros((CHUNK, D), jnp.float32)

  deg2 = _get_sc_degree()(dst_pd, ones1, zeros1)
  h0, nb = _tc_linear(x, W0, b0.reshape(1, D), Wn0, bn0.reshape(1, D))
  m = _tc_scale(nb, deg2)
  s2 = _get_sc_scatter()(src_p, dst_p, m, zeros2)
  return _tc_final(h0, nb, m, s2, deg2, W1, b1.reshape(1, D))


# speedup vs baseline: 1.1096x; 1.0527x over previous
import functools

import jax
import jax.numpy as jnp
from jax import lax
from jax.experimental import pallas as pl
from jax.experimental.pallas import tpu as pltpu
from jax.experimental.pallas import tpu_sc as plsc

N = 10000
E = 320000
D = 128
NC = 2
NS = 16
NW = NC * NS
CHUNK = 64
CW = 160
IH = 32
NBUF = 4
CHUNK_DEG = 128
CW_DEG = 80
IH_DEG = 16
E_PAD = NW * CW * CHUNK
NPAD = E_PAD - E
NDUM = 16
NACC = 10240
ZSTRIPE = NACC // NS
R = 2048

def _sc_degree_body(dst_hbm, ones_hbm, zeros_hbm, out_hbm, idx_v, ones_v, acc):
  c = lax.axis_index("c")
  s = lax.axis_index("s")
  w = c * NS + s
  pltpu.sync_copy(zeros_hbm, acc.at[pl.ds(s * ZSTRIPE, ZSTRIPE)])
  pltpu.sync_copy(ones_hbm, ones_v)
  for p in range(CW_DEG // IH_DEG):
    pltpu.sync_copy(dst_hbm.at[pl.ds(w * CW_DEG + p * IH_DEG, IH_DEG), :],
                    idx_v)
    if p == 0:
      plsc.subcore_barrier()

    def body(i, carry):
      pltpu.sync_copy(ones_v, acc.at[idx_v.at[i]], add=True)
      return carry

    lax.fori_loop(0, IH_DEG, body, 0)
  plsc.subcore_barrier()
  pltpu.sync_copy(acc.at[pl.ds(s * ZSTRIPE, ZSTRIPE)],
                  out_hbm.at[c, pl.ds(s * ZSTRIPE, ZSTRIPE)])


@functools.cache
def _get_sc_degree():
  mesh = plsc.VectorSubcoreMesh(
      core_axis_name="c", subcore_axis_name="s", num_cores=NC,
      num_subcores=NS)
  return pl.kernel(
      _sc_degree_body,
      out_type=jax.ShapeDtypeStruct((NC, NACC), jnp.float32),
      mesh=mesh,
      scratch_types=[
          pltpu.VMEM((IH_DEG, CHUNK_DEG), jnp.int32),
          pltpu.VMEM((CHUNK_DEG,), jnp.float32),
          pltpu.VMEM_SHARED((NACC,), jnp.float32),
      ],
  )


def _sc_scatter_body(src_hbm, dst_hbm, m_hbm, zeros_hbm, out_hbm,
                     src_v, dst_v, rows_v, acc, sem0, sem1, sem2, sem3,
                     sem_ix):
  sems = [sem0, sem1, sem2, sem3]
  c = lax.axis_index("c")
  s = lax.axis_index("s")
  w = c * NS + s
  pltpu.sync_copy(zeros_hbm, rows_v.at[0])
  for z in range(ZSTRIPE // CHUNK):
    pltpu.sync_copy(rows_v.at[0],
                    acc.at[pl.ds(s * ZSTRIPE + z * CHUNK, CHUNK), :])

  base = w * CW
  nph = CW // IH
  pltpu.sync_copy(src_hbm.at[pl.ds(base, IH), :], src_v.at[0])
  pltpu.sync_copy(dst_hbm.at[pl.ds(base, IH), :], dst_v.at[0])
  plsc.subcore_barrier()
  for p in range(nph):
    pb = p % 2
    if p + 1 < nph:
      pltpu.async_copy(src_hbm.at[pl.ds(base + (p + 1) * IH, IH), :],
                       src_v.at[1 - pb], sem_ix)
      pltpu.async_copy(dst_hbm.at[pl.ds(base + (p + 1) * IH, IH), :],
                       dst_v.at[1 - pb], sem_ix)

    for b in range(NBUF):
      pltpu.async_copy(m_hbm.at[src_v.at[pb, b]], rows_v.at[b], sems[b])

    def body(i, carry):
      for b in range(NBUF):
        ci = NBUF * i + b
        pltpu.make_async_copy(m_hbm.at[src_v.at[pb, ci]], rows_v.at[b],
                              sems[b]).wait()
        pltpu.sync_copy(rows_v.at[b], acc.at[dst_v.at[pb, ci]], add=True)

        @pl.when(ci + NBUF < IH)
        def _():
          pltpu.async_copy(m_hbm.at[src_v.at[pb, ci + NBUF]], rows_v.at[b],
                           sems[b])
      return carry

    lax.fori_loop(0, IH // NBUF, body, 0)
    if p + 1 < nph:
      pltpu.make_async_copy(src_hbm.at[pl.ds(base, IH), :],
                            src_v.at[1 - pb], sem_ix).wait()
      pltpu.make_async_copy(dst_hbm.at[pl.ds(base, IH), :],
                            dst_v.at[1 - pb], sem_ix).wait()
  plsc.subcore_barrier()
  pltpu.sync_copy(acc.at[pl.ds(s * ZSTRIPE, ZSTRIPE), :],
                  out_hbm.at[c, pl.ds(s * ZSTRIPE, ZSTRIPE), :])


@functools.cache
def _get_sc_scatter():
  mesh = plsc.VectorSubcoreMesh(
      core_axis_name="c", subcore_axis_name="s", num_cores=NC,
      num_subcores=NS)
  return pl.kernel(
      _sc_scatter_body,
      out_type=jax.ShapeDtypeStruct((NC, NACC, D), jnp.float32),
      mesh=mesh,
      scratch_types=[
          pltpu.VMEM((2, IH, CHUNK), jnp.int32),
          pltpu.VMEM((2, IH, CHUNK), jnp.int32),
          pltpu.VMEM((NBUF, CHUNK, D), jnp.float32),
          pltpu.VMEM_SHARED((NACC, D), jnp.float32),
          pltpu.SemaphoreType.DMA,
          pltpu.SemaphoreType.DMA,
          pltpu.SemaphoreType.DMA,
          pltpu.SemaphoreType.DMA,
          pltpu.SemaphoreType.DMA,
      ],
  )


def _tc_linear_body(x_ref, w0_ref, b0_ref, wn0_ref, bn0_ref,
                    h0_ref, nb_ref):
  xb = x_ref[...]
  h0 = lax.dot_general(xb, w0_ref[...], (((1,), (1,)), ((), ())),
                       preferred_element_type=jnp.float32) + b0_ref[...]
  nb = lax.dot_general(h0, wn0_ref[...], (((1,), (1,)), ((), ())),
                       preferred_element_type=jnp.float32) + bn0_ref[...]
  h0_ref[...] = h0
  nb_ref[...] = nb


def _tc_linear(x, w0, b0, wn0, bn0):
  return pl.pallas_call(
      _tc_linear_body,
      grid=(pl.cdiv(N, R),),
      in_specs=[
          pl.BlockSpec((R, D), lambda i: (i, 0)),
          pl.BlockSpec((D, D), lambda i: (0, 0)),
          pl.BlockSpec((1, D), lambda i: (0, 0)),
          pl.BlockSpec((D, D), lambda i: (0, 0)),
          pl.BlockSpec((1, D), lambda i: (0, 0)),
      ],
      out_specs=[
          pl.BlockSpec((R, D), lambda i: (i, 0)),
          pl.BlockSpec((R, D), lambda i: (i, 0)),
      ],
      out_shape=[
          jax.ShapeDtypeStruct((N, D), jnp.float32),
          jax.ShapeDtypeStruct((N, D), jnp.float32),
      ],
  )(x, w0, b0, wn0, bn0)


def _tc_scale_body(nb_ref, deg_ref, m_ref):
  i = pl.program_id(0)
  deg = deg_ref[0, pl.ds(i * R, R)] + deg_ref[1, pl.ds(i * R, R)] + 1.0
  m_ref[...] = nb_ref[...] * lax.rsqrt(deg)[:, None]


def _tc_scale(nb, deg2):
  return pl.pallas_call(
      _tc_scale_body,
      grid=(pl.cdiv(N, R),),
      in_specs=[
          pl.BlockSpec((R, D), lambda i: (i, 0)),
          pl.BlockSpec((NC, NACC), lambda i: (0, 0)),
      ],
      out_specs=pl.BlockSpec((R, D), lambda i: (i, 0)),
      out_shape=jax.ShapeDtypeStruct((N, D), jnp.float32),
  )(nb, deg2)


def _tc_final_body(h0_ref, nb_ref, s_ref, deg_ref, w1_ref, b1_ref,
                   out_ref):
  i = pl.program_id(0)
  deg = deg_ref[0, pl.ds(i * R, R)] + deg_ref[1, pl.ds(i * R, R)] + 1.0
  dinv = lax.rsqrt(deg)[:, None]
  nb = nb_ref[...]
  agg = dinv * (s_ref[0] + s_ref[1] + dinv * nb)
  h = h0_ref[...] + 0.5 * agg + 0.5 * nb
  nrm = jnp.sqrt(jnp.sum(h * h, axis=1, keepdims=True))
  h = h / jnp.maximum(nrm, 1e-12)
  h = jnp.maximum(h, 0.0)
  out_ref[...] = lax.dot_general(h, w1_ref[...], (((1,), (1,)), ((), ())),
                                 preferred_element_type=jnp.float32
                                 ) + b1_ref[...]


def _tc_final(h0, nb, s2, deg2, w1, b1):
  return pl.pallas_call(
      _tc_final_body,
      grid=(pl.cdiv(N, R),),
      in_specs=[
          pl.BlockSpec((R, D), lambda i: (i, 0)),
          pl.BlockSpec((R, D), lambda i: (i, 0)),
          pl.BlockSpec((NC, R, D), lambda i: (0, i, 0)),
          pl.BlockSpec((NC, NACC), lambda i: (0, 0)),
          pl.BlockSpec((D, D), lambda i: (0, 0)),
          pl.BlockSpec((1, D), lambda i: (0, 0)),
      ],
      out_specs=pl.BlockSpec((R, D), lambda i: (i, 0)),
      out_shape=jax.ShapeDtypeStruct((N, D), jnp.float32),
  )(h0, nb, s2, deg2, w1, b1)


@jax.jit
def kernel(x, edge_index, W0, b0, Wn0, bn0, W1, b1):
  src = edge_index[0].astype(jnp.int32)
  dst = edge_index[1].astype(jnp.int32)
  ar = jnp.arange(NPAD, dtype=jnp.int32)
  src_p = jnp.concatenate([src, ar % N]).reshape(NW * CW, CHUNK)
  dst_f = jnp.concatenate([dst, N + (ar % NDUM)])
  dst_p = dst_f.reshape(NW * CW, CHUNK)
  dst_pd = dst_f.reshape(NW * CW_DEG, CHUNK_DEG)
  ones1 = jnp.ones((CHUNK_DEG,), jnp.float32)
  zeros1 = jnp.zeros((ZSTRIPE,), jnp.float32)
  zeros2 = jnp.zeros((CHUNK, D), jnp.float32)

  deg2 = _get_sc_degree()(dst_pd, ones1, zeros1)
  h0, nb = _tc_linear(x, W0, b0.reshape(1, D), Wn0, bn0.reshape(1, D))
  m = _tc_scale(nb, deg2)
  s2 = _get_sc_scatter()(src_p, dst_p, m, zeros2)
  return _tc_final(h0, nb, s2, deg2, W1, b1.reshape(1, D))
